# Initial kernel scaffold; baseline (speedup 1.0000x reference)
import functools

import jax
import jax.numpy as jnp
from jax import lax
from jax.experimental import pallas as pl
from jax.experimental.pallas import tpu as pltpu
from jax.experimental.pallas import tpu_sc as plsc

N = 10000
D = 128
E = 320000
NC = 2
NS = 16
NW = NC * NS
CH = 128
EPW = E // NW
NCH = 80
EPWP = NCH * CH
NPAD = 10240
DUMMY = NPAD - 1
RPT = NPAD // NS
BM = 1024
GRID = NPAD // BM

_mesh = plsc.VectorSubcoreMesh(core_axis_name="c", subcore_axis_name="s")


def _zero16():
    return jnp.zeros((16,), jnp.float32)


def _sc_degree(dstp):
    @functools.partial(
        pl.kernel,
        out_type=jax.ShapeDtypeStruct((NC, NCH, D), jnp.float32),
        mesh=_mesh,
        scratch_types=[
            pltpu.VMEM((NCH, CH), jnp.int32),
            pltpu.VMEM((NCH, D), jnp.float32),
            pltpu.VMEM((NCH,), jnp.int32),
            pltpu.VMEM((NCH // NS, D), jnp.float32),
            pltpu.VMEM_SHARED((NCH, D), jnp.float32),
        ],
    )
    def k(dst_hbm, out_hbm, dst_v, cnt, idv, zb, acc):
        c = lax.axis_index("c")
        s = lax.axis_index("s")
        wid = c * NS + s
        pltpu.sync_copy(dst_hbm.at[wid], dst_v)
        zrows = NCH // NS

        @pl.loop(0, NCH)
        def _(r):
            for kk in range(D // 16):
                cnt[r, pl.ds(kk * 16, 16)] = _zero16()

        for r in range(zrows):
            for kk in range(D // 16):
                zb[r, pl.ds(kk * 16, 16)] = _zero16()
        for kk in range(NCH // 16):
            idv[pl.ds(kk * 16, 16)] = lax.iota(jnp.int32, 16) + kk * 16
        pltpu.sync_copy(zb, acc.at[pl.ds(s * zrows, zrows)])
        plsc.subcore_barrier()

        ones = jnp.ones((16,), jnp.float32)

        @pl.loop(0, NCH)
        def _(j):
            for kk in range(CH // 16):
                v = dst_v[j, pl.ds(kk * 16, 16)]
                plsc.addupdate_scatter(cnt, [v >> 7, v & 127], ones)

        pltpu.sync_copy(cnt, acc.at[idv], add=True)
        plsc.subcore_barrier()
        pltpu.sync_copy(acc.at[pl.ds(s * zrows, zrows)],
                        out_hbm.at[c, pl.ds(s * zrows, zrows)])

    return k(dstp)


def _sc_scatter(g, srcp, dstp):
    @functools.partial(
        pl.kernel,
        out_type=jax.ShapeDtypeStruct((NC, NPAD, D), jnp.float32),
        mesh=_mesh,
        scratch_types=[
            pltpu.VMEM((NCH, CH), jnp.int32),
            pltpu.VMEM((NCH, CH), jnp.int32),
            pltpu.VMEM((CH, D), jnp.float32),
            pltpu.VMEM((CH, D), jnp.float32),
            pltpu.VMEM_SHARED((NPAD, D), jnp.float32),
            pltpu.SemaphoreType.DMA,
            pltpu.SemaphoreType.DMA,
        ],
    )
    def k(g_hbm, src_hbm, dst_hbm, out_hbm, src_v, dst_v, buf0, buf1, acc,
          sem0, sem1):
        c = lax.axis_index("c")
        s = lax.axis_index("s")
        wid = c * NS + s
        pltpu.sync_copy(src_hbm.at[wid], src_v)
        pltpu.sync_copy(dst_hbm.at[wid], dst_v)

        @pl.loop(0, CH)
        def _(r):
            for kk in range(D // 16):
                buf0[r, pl.ds(kk * 16, 16)] = _zero16()

        for kk in range(RPT // CH):
            pltpu.sync_copy(buf0, acc.at[pl.ds(s * RPT + kk * CH, CH)])
        plsc.subcore_barrier()

        def gather(j, buf, sem):
            return pltpu.make_async_copy(g_hbm.at[src_v.at[j]], buf, sem)

        gather(0, buf0, sem0).start()
        gather(1, buf1, sem1).start()

        @pl.loop(0, NCH // 2)
        def _(p):
            j0 = 2 * p
            j1 = j0 + 1
            gather(j0, buf0, sem0).wait()
            pltpu.sync_copy(buf0, acc.at[dst_v.at[j0]], add=True)

            @pl.when(p < NCH // 2 - 1)
            def _():
                gather(j0 + 2, buf0, sem0).start()

            gather(j1, buf1, sem1).wait()
            pltpu.sync_copy(buf1, acc.at[dst_v.at[j1]], add=True)

            @pl.when(p < NCH // 2 - 1)
            def _():
                gather(j1 + 2, buf1, sem1).start()

        plsc.subcore_barrier()
        for kk in range(RPT // CH):
            pltpu.sync_copy(acc.at[pl.ds(s * RPT + kk * CH, CH)],
                            out_hbm.at[c, pl.ds(s * RPT + kk * CH, CH)])

    return k(g, srcp, dstp)


def _row_spec():
    return pl.BlockSpec((BM, D), lambda i: (i, 0))


def _col_spec():
    return pl.BlockSpec((BM, 1), lambda i: (i, 0))


def _full_spec(shape):
    return pl.BlockSpec(shape, lambda i: tuple(0 for _ in shape))


def _tc_first(x, w1, cnt_col):
    def body(x_ref, w_ref, cnt_ref, g_ref, dis_ref):
        dis = lax.rsqrt(cnt_ref[...] + 1.0)
        h = jnp.dot(x_ref[...], w_ref[...], preferred_element_type=jnp.float32)
        g_ref[...] = h * dis
        dis_ref[...] = dis

    return pl.pallas_call(
        body,
        grid=(GRID,),
        in_specs=[_row_spec(), _full_spec((D, D)), _col_spec()],
        out_specs=[_row_spec(), _col_spec()],
        out_shape=[
            jax.ShapeDtypeStruct((NPAD, D), jnp.float32),
            jax.ShapeDtypeStruct((NPAD, 1), jnp.float32),
        ],
    )(x, w1, cnt_col)


def _tc_mid(agg0, agg1, g, dis, b, w):
    def body(a0_ref, a1_ref, g_ref, dis_ref, b_ref, w_ref, o_ref):
        dis = dis_ref[...]
        t = (a0_ref[...] + a1_ref[...] + g_ref[...]) * dis + b_ref[...]
        t = jnp.maximum(t, 0.0)
        o_ref[...] = jnp.dot(t, w_ref[...],
                             preferred_element_type=jnp.float32) * dis

    return pl.pallas_call(
        body,
        grid=(GRID,),
        in_specs=[_row_spec(), _row_spec(), _row_spec(), _col_spec(),
                  _full_spec((D,)), _full_spec((D, D))],
        out_specs=_row_spec(),
        out_shape=jax.ShapeDtypeStruct((NPAD, D), jnp.float32),
    )(agg0, agg1, g, dis, b, w)


def _tc_last(agg0, agg1, g, dis, b):
    def body(a0_ref, a1_ref, g_ref, dis_ref, b_ref, o_ref):
        o_ref[...] = ((a0_ref[...] + a1_ref[...] + g_ref[...]) * dis_ref[...]
                      + b_ref[...])

    return pl.pallas_call(
        body,
        grid=(GRID,),
        in_specs=[_row_spec(), _row_spec(), _row_spec(), _col_spec(),
                  _full_spec((D,))],
        out_specs=_row_spec(),
        out_shape=jax.ShapeDtypeStruct((NPAD, D), jnp.float32),
    )(agg0, agg1, g, dis, b)


def kernel(x, edge_index, W1, b1, W2, b2, W3, b3):
    x_pad = jnp.pad(x, ((0, NPAD - N), (0, 0)))
    pad_cols = jnp.full((NW, EPWP - EPW), DUMMY, jnp.int32)
    srcp = jnp.concatenate(
        [edge_index[0].reshape(NW, EPW), pad_cols], axis=1).reshape(NW, NCH, CH)
    dstp = jnp.concatenate(
        [edge_index[1].reshape(NW, EPW), pad_cols], axis=1).reshape(NW, NCH, CH)

    cnt = _sc_degree(dstp)
    cnt_col = (cnt[0] + cnt[1]).reshape(NPAD, 1)

    g1, dis = _tc_first(x_pad, W1, cnt_col)
    agg = _sc_scatter(g1, srcp, dstp)
    g2 = _tc_mid(agg[0], agg[1], g1, dis, b1, W2)
    agg = _sc_scatter(g2, srcp, dstp)
    g3 = _tc_mid(agg[0], agg[1], g2, dis, b2, W3)
    agg = _sc_scatter(g3, srcp, dstp)
    out = _tc_last(agg[0], agg[1], g3, dis, b3)
    return out[:N]


# speedup vs baseline: 9.3893x; 9.3893x over previous
import functools

import jax
import jax.numpy as jnp
from jax import lax
from jax.experimental import pallas as pl
from jax.experimental.pallas import tpu as pltpu
from jax.experimental.pallas import tpu_sc as plsc

N = 10000
D = 128
E = 320000
NC = 2
NS = 16
NW = NC * NS
CH = 128
EPW = E // NW
NCH = 80
NPH = 2
CPP = NCH // NPH
EPWP = NCH * CH
NPAD = 10240
DUMMY = NPAD - 1
RPT = NPAD // NS
HR = NPAD // D
BM = 1024
GRID = NPAD // BM

_mesh = plsc.VectorSubcoreMesh(core_axis_name="c", subcore_axis_name="s")


def _zero16():
    return jnp.zeros((16,), jnp.float32)


def _sc_degree(dstp):
    @functools.partial(
        pl.kernel,
        out_type=jax.ShapeDtypeStruct((NC, HR, D), jnp.float32),
        mesh=_mesh,
        scratch_types=[
            pltpu.VMEM((CPP, CH), jnp.int32),
            pltpu.VMEM((HR, D), jnp.float32),
            pltpu.VMEM((HR,), jnp.int32),
            pltpu.VMEM((8, D), jnp.float32),
            pltpu.VMEM_SHARED((HR, D), jnp.float32),
        ],
        compiler_params=pltpu.CompilerParams(needs_layout_passes=False),
    )
    def k(dst_hbm, out_hbm, dst_v, cnt, idv, zb, acc):
        c = lax.axis_index("c")
        s = lax.axis_index("s")
        wid = c * NS + s
        zrows = 8

        @pl.loop(0, HR)
        def _(r):
            for kk in range(D // 16):
                cnt[r, pl.ds(kk * 16, 16)] = _zero16()

        for r in range(zrows):
            for kk in range(D // 16):
                zb[r, pl.ds(kk * 16, 16)] = _zero16()
        for kk in range(HR // 16):
            idv[pl.ds(kk * 16, 16)] = lax.iota(jnp.int32, 16) + kk * 16

        @pl.when(s < HR // zrows)
        def _():
            pltpu.sync_copy(zb, acc.at[pl.ds(s * zrows, zrows)])

        plsc.subcore_barrier()

        ones = jnp.ones((16,), jnp.float32)

        for ph in range(NPH):
            pltpu.sync_copy(dst_hbm.at[wid, pl.ds(ph * CPP, CPP)], dst_v)

            @pl.loop(0, CPP)
            def _(j):
                for kk in range(CH // 16):
                    v = dst_v[j, pl.ds(kk * 16, 16)]
                    plsc.addupdate_scatter(cnt, [v >> 7, v & 127], ones)

        pltpu.sync_copy(cnt, acc.at[idv], add=True)
        plsc.subcore_barrier()

        @pl.when(s < HR // zrows)
        def _():
            pltpu.sync_copy(acc.at[pl.ds(s * zrows, zrows)],
                            out_hbm.at[c, pl.ds(s * zrows, zrows)])

    return k(dstp)


def _sc_scatter(g, srcp, dstp):
    @functools.partial(
        pl.kernel,
        out_type=jax.ShapeDtypeStruct((NC, NPAD, D), jnp.float32),
        mesh=_mesh,
        scratch_types=[
            pltpu.VMEM((CPP, CH), jnp.int32),
            pltpu.VMEM((CPP, CH), jnp.int32),
            pltpu.VMEM((CH, D), jnp.float32),
            pltpu.VMEM((CH, D), jnp.float32),
            pltpu.VMEM_SHARED((NPAD, D), jnp.float32),
            pltpu.SemaphoreType.DMA,
            pltpu.SemaphoreType.DMA,
        ],
    )
    def k(g_hbm, src_hbm, dst_hbm, out_hbm, src_v, dst_v, buf0, buf1, acc,
          sem0, sem1):
        c = lax.axis_index("c")
        s = lax.axis_index("s")
        wid = c * NS + s

        @pl.loop(0, CH)
        def _(r):
            for kk in range(D // 16):
                buf0[r, pl.ds(kk * 16, 16)] = _zero16()

        for kk in range(RPT // CH):
            pltpu.sync_copy(buf0, acc.at[pl.ds(s * RPT + kk * CH, CH)])
        plsc.subcore_barrier()

        def gather(j, buf, sem):
            return pltpu.make_async_copy(g_hbm.at[src_v.at[j]], buf, sem)

        for ph in range(NPH):
            pltpu.sync_copy(src_hbm.at[wid, pl.ds(ph * CPP, CPP)], src_v)
            pltpu.sync_copy(dst_hbm.at[wid, pl.ds(ph * CPP, CPP)], dst_v)
            gather(0, buf0, sem0).start()
            gather(1, buf1, sem1).start()

            @pl.loop(0, CPP // 2)
            def _(p):
                j0 = 2 * p
                j1 = j0 + 1
                gather(j0, buf0, sem0).wait()
                pltpu.sync_copy(buf0, acc.at[dst_v.at[j0]], add=True)

                @pl.when(p < CPP // 2 - 1)
                def _():
                    gather(j0 + 2, buf0, sem0).start()

                gather(j1, buf1, sem1).wait()
                pltpu.sync_copy(buf1, acc.at[dst_v.at[j1]], add=True)

                @pl.when(p < CPP // 2 - 1)
                def _():
                    gather(j1 + 2, buf1, sem1).start()

        plsc.subcore_barrier()
        for kk in range(RPT // CH):
            pltpu.sync_copy(acc.at[pl.ds(s * RPT + kk * CH, CH)],
                            out_hbm.at[c, pl.ds(s * RPT + kk * CH, CH)])

    return k(g, srcp, dstp)


def _row_spec():
    return pl.BlockSpec((BM, D), lambda i: (i, 0))


def _col_spec():
    return pl.BlockSpec((BM, 1), lambda i: (i, 0))


def _full_spec(shape):
    return pl.BlockSpec(shape, lambda i: tuple(0 for _ in shape))


def _tc_first(x, w1, cnt_col):
    def body(x_ref, w_ref, cnt_ref, g_ref, dis_ref):
        dis = lax.rsqrt(cnt_ref[...] + 1.0)
        h = jnp.dot(x_ref[...], w_ref[...], preferred_element_type=jnp.float32)
        g_ref[...] = h * dis
        dis_ref[...] = dis

    return pl.pallas_call(
        body,
        grid=(GRID,),
        in_specs=[_row_spec(), _full_spec((D, D)), _col_spec()],
        out_specs=[_row_spec(), _col_spec()],
        out_shape=[
            jax.ShapeDtypeStruct((NPAD, D), jnp.float32),
            jax.ShapeDtypeStruct((NPAD, 1), jnp.float32),
        ],
    )(x, w1, cnt_col)


def _tc_mid(agg0, agg1, g, dis, b, w):
    def body(a0_ref, a1_ref, g_ref, dis_ref, b_ref, w_ref, o_ref):
        dis = dis_ref[...]
        t = (a0_ref[...] + a1_ref[...] + g_ref[...]) * dis + b_ref[...]
        t = jnp.maximum(t, 0.0)
        o_ref[...] = jnp.dot(t, w_ref[...],
                             preferred_element_type=jnp.float32) * dis

    return pl.pallas_call(
        body,
        grid=(GRID,),
        in_specs=[_row_spec(), _row_spec(), _row_spec(), _col_spec(),
                  _full_spec((D,)), _full_spec((D, D))],
        out_specs=_row_spec(),
        out_shape=jax.ShapeDtypeStruct((NPAD, D), jnp.float32),
    )(agg0, agg1, g, dis, b, w)


def _tc_last(agg0, agg1, g, dis, b):
    def body(a0_ref, a1_ref, g_ref, dis_ref, b_ref, o_ref):
        o_ref[...] = ((a0_ref[...] + a1_ref[...] + g_ref[...]) * dis_ref[...]
                      + b_ref[...])

    return pl.pallas_call(
        body,
        grid=(GRID,),
        in_specs=[_row_spec(), _row_spec(), _row_spec(), _col_spec(),
                  _full_spec((D,))],
        out_specs=_row_spec(),
        out_shape=jax.ShapeDtypeStruct((NPAD, D), jnp.float32),
    )(agg0, agg1, g, dis, b)


def kernel(x, edge_index, W1, b1, W2, b2, W3, b3):
    x_pad = jnp.pad(x, ((0, NPAD - N), (0, 0)))
    pad_cols = jnp.full((NW, EPWP - EPW), DUMMY, jnp.int32)
    srcp = jnp.concatenate(
        [edge_index[0].reshape(NW, EPW), pad_cols], axis=1).reshape(NW, NCH, CH)
    dstp = jnp.concatenate(
        [edge_index[1].reshape(NW, EPW), pad_cols], axis=1).reshape(NW, NCH, CH)

    cnt = _sc_degree(dstp)
    cnt_col = (cnt[0] + cnt[1]).reshape(NPAD, 1)

    g1, dis = _tc_first(x_pad, W1, cnt_col)
    agg = _sc_scatter(g1, srcp, dstp)
    g2 = _tc_mid(agg[0], agg[1], g1, dis, b1, W2)
    agg = _sc_scatter(g2, srcp, dstp)
    g3 = _tc_mid(agg[0], agg[1], g2, dis, b2, W3)
    agg = _sc_scatter(g3, srcp, dstp)
    out = _tc_last(agg[0], agg[1], g3, dis, b3)
    return out[:N]
